# baseline (speedup 1.0000x reference)
import functools

import jax
import jax.numpy as jnp
from jax import lax
from jax.experimental import pallas as pl
from jax.experimental.pallas import tpu as pltpu
from jax.experimental.pallas import tpu_sc as plsc

EMBED_DIM = 64
IDX_MINOR = 128
CHUNK = 2
NB = 6
K = 4
NC = 2
NS = 16
NW = NC * NS

TAB_BLK = 40960
OUT_BLK = 4096
SLABS_PER_STEP = 10


def _build_lookup(n_idx, vocab):
    assert n_idx % (NW * IDX_MINOR) == 0
    rows_total = n_idx // IDX_MINOR
    rows_per_w = rows_total // NW
    assert rows_per_w % CHUNK == 0
    n_chunks = rows_per_w // CHUNK
    assert n_chunks > NB > K
    ci = CHUNK * IDX_MINOR

    mesh = plsc.VectorSubcoreMesh(
        core_axis_name="c", subcore_axis_name="s", num_cores=NC
    )

    @functools.partial(
        pl.kernel,
        mesh=mesh,
        out_type=jax.ShapeDtypeStruct((n_idx, EMBED_DIM), jnp.float32),
        scratch_types=[
            pltpu.VMEM((rows_per_w, IDX_MINOR), jnp.int32),
            pltpu.VMEM((NB, ci, EMBED_DIM), jnp.float32),
            pltpu.SemaphoreType.DMA((NB,)),
            pltpu.SemaphoreType.DMA((NB,)),
        ],
        compiler_params=pltpu.CompilerParams(use_tc_tiling_on_sc=False),
    )
    def lookup(idx_hbm, tab_hbm, out_hbm, idx_all, rows_v, gsem, wsem):
        wid = lax.axis_index("s") * NC + lax.axis_index("c")
        row0 = wid * rows_per_w
        out0 = row0 * IDX_MINOR

        pltpu.sync_copy(idx_hbm.at[pl.ds(row0, rows_per_w)], idx_all)

        def fire_gathers(c, b):
            for i in range(CHUNK):
                pltpu.async_copy(
                    tab_hbm.at[idx_all.at[c * CHUNK + i]],
                    rows_v.at[b, pl.ds(i * IDX_MINOR, IDX_MINOR)],
                    gsem.at[b],
                )

        def wait_gathers(b):
            pltpu.make_async_copy(
                tab_hbm.at[pl.ds(0, ci)], rows_v.at[b], gsem.at[b]
            ).wait()

        def fire_write(c, b):
            pltpu.async_copy(
                rows_v.at[b], out_hbm.at[pl.ds(out0 + c * ci, ci)], wsem.at[b]
            )

        def wait_write(b):
            pltpu.make_async_copy(
                rows_v.at[b], out_hbm.at[pl.ds(0, ci)], wsem.at[b]
            ).wait()

        for c in range(K):
            fire_gathers(c, c)
        for c in range(K, NB):
            fire_gathers(c, c)
            wait_gathers(c - K)
            fire_write(c - K, c - K)

        def body(c, _):
            b = lax.rem(c, NB)
            br = lax.rem(c - K, NB)
            wait_write(b)
            fire_gathers(c, b)
            wait_gathers(br)
            fire_write(c - K, br)
            return _

        lax.fori_loop(NB, n_chunks, body, None)

        for c in range(n_chunks - K, n_chunks):
            wait_gathers(c % NB)
            fire_write(c, c % NB)
        for c in range(n_chunks - NB, n_chunks):
            wait_write(c % NB)

    return lookup


def _eye(n, dtype):
    r = lax.broadcasted_iota(jnp.int32, (n, n), 0)
    c = lax.broadcasted_iota(jnp.int32, (n, n), 1)
    return (r == c).astype(dtype)


def _tab_transpose_kernel(x_ref, o_ref):
    dim = x_ref.shape[0]
    h = TAB_BLK // 2
    ident = _eye(dim, x_ref.dtype)
    t = lax.dot_general(
        x_ref[...], ident, (((0,), (0,)), ((), ())),
        preferred_element_type=jnp.float32,
    )
    o_ref[:, 0:dim] = lax.slice_in_dim(t, 0, h, axis=0)
    o_ref[:, dim:2 * dim] = lax.slice_in_dim(t, h, TAB_BLK, axis=0)


def _transpose_table(tab_t):
    dim, vocab = tab_t.shape
    n_blocks = pl.cdiv(vocab, TAB_BLK)
    return pl.pallas_call(
        _tab_transpose_kernel,
        grid=(n_blocks,),
        in_specs=[pl.BlockSpec((dim, TAB_BLK), lambda i: (0, i))],
        out_specs=pl.BlockSpec((TAB_BLK // 2, 2 * dim), lambda i: (i, 0)),
        out_shape=jax.ShapeDtypeStruct(
            (n_blocks * TAB_BLK // 2, 2 * dim), tab_t.dtype
        ),
    )(tab_t)


def _out_transpose_kernel(x_ref, o_ref):
    dim = o_ref.shape[0] // SLABS_PER_STEP
    half = OUT_BLK // 2
    r = lax.broadcasted_iota(jnp.int32, (dim, 2 * dim), 0)
    c = lax.broadcasted_iota(jnp.int32, (dim, 2 * dim), 1)
    e0 = (c == r).astype(x_ref.dtype)
    e1 = (c == r + dim).astype(x_ref.dtype)
    for j in range(SLABS_PER_STEP):
        pj = lax.slice_in_dim(x_ref[...], j * half, (j + 1) * half, axis=0)
        for e, lo in ((e0, 0), (e1, half)):
            o_ref[pl.ds(j * dim, dim), pl.ds(lo, half)] = lax.dot_general(
                e, pj, (((1,), (1,)), ((), ())),
                preferred_element_type=jnp.float32,
            )


def _transpose_out(rows_pairs, n_slabs, dim):
    assert rows_pairs.shape == (n_slabs * OUT_BLK // 2, 2 * dim)
    assert n_slabs % SLABS_PER_STEP == 0
    return pl.pallas_call(
        _out_transpose_kernel,
        grid=(n_slabs // SLABS_PER_STEP,),
        in_specs=[
            pl.BlockSpec((SLABS_PER_STEP * OUT_BLK // 2, 2 * dim), lambda i: (i, 0))
        ],
        out_specs=pl.BlockSpec((SLABS_PER_STEP * dim, OUT_BLK), lambda i: (i, 0)),
        out_shape=jax.ShapeDtypeStruct((n_slabs * dim, OUT_BLK), rows_pairs.dtype),
    )(rows_pairs)


_LOOKUP = None


def kernel(x, table):
    global _LOOKUP
    batch, seq = x.shape
    n_idx = x.size
    if _LOOKUP is None:
        _LOOKUP = _build_lookup(n_idx, table.shape[0])
    vocab = table.shape[0]
    xt = x.T.astype(jnp.int32).reshape(seq, 2, batch // 2)
    xi = xt.transpose(0, 2, 1).reshape(-1)
    q = xi % TAB_BLK
    h = TAB_BLK // 2
    xi = (xi - q) + jnp.where(q < h, 2 * q, 2 * (q - h) + 1)
    idx2d = xi.reshape(n_idx // IDX_MINOR, IDX_MINOR)
    tab_p = _transpose_table(table.T)
    tab_rm = tab_p.reshape(tab_p.shape[0] * 2, EMBED_DIM)
    rows = _LOOKUP(idx2d, tab_rm)
    ot = _transpose_out(
        rows.reshape(n_idx // 2, 2 * EMBED_DIM), n_slabs=seq, dim=EMBED_DIM
    )
    ot3 = ot.reshape(seq, EMBED_DIM, batch)
    return ot3.transpose(2, 0, 1)


# speedup vs baseline: 1.0066x; 1.0066x over previous
import functools

import jax
import jax.numpy as jnp
from jax import lax
from jax.experimental import pallas as pl
from jax.experimental.pallas import tpu as pltpu
from jax.experimental.pallas import tpu_sc as plsc

EMBED_DIM = 64
IDX_MINOR = 128
CHUNK = 2
NB = 6
K = 4
NC = 2
NS = 16
NW = NC * NS

TAB_BLK = 32768
OUT_BLK = 4096
SLABS_PER_STEP = 8


def _build_lookup(n_idx, vocab):
    assert n_idx % (NW * IDX_MINOR) == 0
    rows_total = n_idx // IDX_MINOR
    rows_per_w = rows_total // NW
    assert rows_per_w % CHUNK == 0
    n_chunks = rows_per_w // CHUNK
    assert n_chunks > NB > K
    ci = CHUNK * IDX_MINOR

    mesh = plsc.VectorSubcoreMesh(
        core_axis_name="c", subcore_axis_name="s", num_cores=NC
    )

    @functools.partial(
        pl.kernel,
        mesh=mesh,
        out_type=jax.ShapeDtypeStruct((n_idx, EMBED_DIM), jnp.float32),
        scratch_types=[
            pltpu.VMEM((rows_per_w, IDX_MINOR), jnp.int32),
            pltpu.VMEM((NB, ci, EMBED_DIM), jnp.float32),
            pltpu.SemaphoreType.DMA((NB,)),
            pltpu.SemaphoreType.DMA((NB,)),
        ],
        compiler_params=pltpu.CompilerParams(use_tc_tiling_on_sc=False),
    )
    def lookup(idx_hbm, tab_hbm, out_hbm, idx_all, rows_v, gsem, wsem):
        wid = lax.axis_index("s") * NC + lax.axis_index("c")
        row0 = wid * rows_per_w
        out0 = row0 * IDX_MINOR

        pltpu.sync_copy(idx_hbm.at[pl.ds(row0, rows_per_w)], idx_all)

        def fire_gathers(c, b):
            for i in range(CHUNK):
                pltpu.async_copy(
                    tab_hbm.at[idx_all.at[c * CHUNK + i]],
                    rows_v.at[b, pl.ds(i * IDX_MINOR, IDX_MINOR)],
                    gsem.at[b],
                )

        def wait_gathers(b):
            pltpu.make_async_copy(
                tab_hbm.at[pl.ds(0, ci)], rows_v.at[b], gsem.at[b]
            ).wait()

        def fire_write(c, b):
            pltpu.async_copy(
                rows_v.at[b], out_hbm.at[pl.ds(out0 + c * ci, ci)], wsem.at[b]
            )

        def wait_write(b):
            pltpu.make_async_copy(
                rows_v.at[b], out_hbm.at[pl.ds(0, ci)], wsem.at[b]
            ).wait()

        for c in range(K):
            fire_gathers(c, c)
        for c in range(K, NB):
            fire_gathers(c, c)
            wait_gathers(c - K)
            fire_write(c - K, c - K)

        def body(c, _):
            b = lax.rem(c, NB)
            br = lax.rem(c - K, NB)
            wait_write(b)
            fire_gathers(c, b)
            wait_gathers(br)
            fire_write(c - K, br)
            return _

        lax.fori_loop(NB, n_chunks, body, None)

        for c in range(n_chunks - K, n_chunks):
            wait_gathers(c % NB)
            fire_write(c, c % NB)
        for c in range(n_chunks - NB, n_chunks):
            wait_write(c % NB)

    return lookup


def _eye(n, dtype):
    r = lax.broadcasted_iota(jnp.int32, (n, n), 0)
    c = lax.broadcasted_iota(jnp.int32, (n, n), 1)
    return (r == c).astype(dtype)


def _tab_transpose_kernel(x_ref, o_ref):
    dim = x_ref.shape[0]
    h = TAB_BLK // 2
    ident = _eye(dim, x_ref.dtype)
    t = lax.dot_general(
        x_ref[...], ident, (((0,), (0,)), ((), ())),
        preferred_element_type=jnp.float32,
    )
    o_ref[:, 0:dim] = lax.slice_in_dim(t, 0, h, axis=0)
    o_ref[:, dim:2 * dim] = lax.slice_in_dim(t, h, TAB_BLK, axis=0)


def _transpose_table(tab_t):
    dim, vocab = tab_t.shape
    n_blocks = pl.cdiv(vocab, TAB_BLK)
    return pl.pallas_call(
        _tab_transpose_kernel,
        grid=(n_blocks,),
        in_specs=[pl.BlockSpec((dim, TAB_BLK), lambda i: (0, i))],
        out_specs=pl.BlockSpec((TAB_BLK // 2, 2 * dim), lambda i: (i, 0)),
        out_shape=jax.ShapeDtypeStruct(
            (n_blocks * TAB_BLK // 2, 2 * dim), tab_t.dtype
        ),
    )(tab_t)


def _out_transpose_kernel(x_ref, o_ref):
    dim = o_ref.shape[0] // SLABS_PER_STEP
    half = OUT_BLK // 2
    r = lax.broadcasted_iota(jnp.int32, (dim, 2 * dim), 0)
    c = lax.broadcasted_iota(jnp.int32, (dim, 2 * dim), 1)
    e0 = (c == r).astype(x_ref.dtype)
    e1 = (c == r + dim).astype(x_ref.dtype)
    for j in range(SLABS_PER_STEP):
        pj = lax.slice_in_dim(x_ref[...], j * half, (j + 1) * half, axis=0)
        for e, lo in ((e0, 0), (e1, half)):
            o_ref[pl.ds(j * dim, dim), pl.ds(lo, half)] = lax.dot_general(
                e, pj, (((1,), (1,)), ((), ())),
                preferred_element_type=jnp.float32,
            )


def _transpose_out(rows_pairs, n_slabs, dim):
    assert rows_pairs.shape == (n_slabs * OUT_BLK // 2, 2 * dim)
    assert n_slabs % SLABS_PER_STEP == 0
    return pl.pallas_call(
        _out_transpose_kernel,
        grid=(n_slabs // SLABS_PER_STEP,),
        in_specs=[
            pl.BlockSpec((SLABS_PER_STEP * OUT_BLK // 2, 2 * dim), lambda i: (i, 0))
        ],
        out_specs=pl.BlockSpec((SLABS_PER_STEP * dim, OUT_BLK), lambda i: (i, 0)),
        out_shape=jax.ShapeDtypeStruct((n_slabs * dim, OUT_BLK), rows_pairs.dtype),
    )(rows_pairs)


_LOOKUP = None


def kernel(x, table):
    global _LOOKUP
    batch, seq = x.shape
    n_idx = x.size
    if _LOOKUP is None:
        _LOOKUP = _build_lookup(n_idx, table.shape[0])
    vocab = table.shape[0]
    xt = x.T.astype(jnp.int32).reshape(seq, 2, batch // 2)
    xi = xt.transpose(0, 2, 1).reshape(-1)
    q = xi % TAB_BLK
    h = TAB_BLK // 2
    xi = (xi - q) + jnp.where(q < h, 2 * q, 2 * (q - h) + 1)
    idx2d = xi.reshape(n_idx // IDX_MINOR, IDX_MINOR)
    tab_p = _transpose_table(table.T)
    tab_rm = tab_p.reshape(tab_p.shape[0] * 2, EMBED_DIM)
    rows = _LOOKUP(idx2d, tab_rm)
    ot = _transpose_out(
        rows.reshape(n_idx // 2, 2 * EMBED_DIM), n_slabs=seq, dim=EMBED_DIM
    )
    ot3 = ot.reshape(seq, EMBED_DIM, batch)
    return ot3.transpose(2, 0, 1)
